# Initial kernel scaffold; baseline (speedup 1.0000x reference)
import jax
import jax.numpy as jnp
from jax.experimental import pallas as pl


def kernel(des, tweet, num_prop, cat_prop, edge_index, edge_type, W_cat, b_cat, W_in, b_in, W_rel, W_root, b_rgcn, W_o1, b_o1, W_o2, b_o2):
    raise NotImplementedError("write your pallas kernel here")


# speedup vs baseline: 3.2887x; 3.2887x over previous
import jax
import jax.numpy as jnp
from jax import lax
from jax.experimental import pallas as pl
from jax.experimental.pallas import tpu as pltpu
from jax.experimental.pallas import tpu_sc as plsc

N = 10000
D = 128
NREL = 2
NROW = 10112
NCORE = 2
NSUB = 16
BLK = 128
SUP = 16
DUMMY = N
RPT = NROW // NSUB


def _leaky(x):
    return jnp.where(x >= 0, x, 0.01 * x)


def _zero_acc(zbuf, acc, sid, width):
    del width
    base = sid * RPT
    for k in range(RPT // 64):
        pltpu.sync_copy(zbuf, acc.at[pl.ds(base + k * 64, 64)])
    rem = RPT % 64
    if rem:
        pltpu.sync_copy(zbuf.at[pl.ds(0, rem)],
                        acc.at[pl.ds(base + (RPT // 64) * 64, rem)])


def _make_edge_kernel(n_sup):
    rows_per_tile = n_sup * SUP
    mesh = plsc.VectorSubcoreMesh(core_axis_name="c", subcore_axis_name="s")

    def body(x, srch, dsth, eth, sp, acc, srcb, dstb, etb, sidxb, rows, zbuf,
             sem):
        cid = lax.axis_index("c")
        sid = lax.axis_index("s")

        def _fill(i, carry):
            for g in range(D // 16):
                zbuf[i, pl.ds(g * 16, 16)] = jnp.zeros((16,), jnp.float32)
            return carry
        lax.fori_loop(0, 64, _fill, 0)

        _zero_acc(zbuf, acc, sid, D)
        plsc.subcore_barrier()

        base0 = sid * rows_per_tile

        def _sup(t, carry):
            base = base0 + t * SUP
            pltpu.sync_copy(srch.at[pl.ds(base, SUP)], srcb)
            pltpu.sync_copy(dsth.at[pl.ds(base, SUP)], dstb)
            pltpu.sync_copy(eth.at[pl.ds(base, SUP)], etb)

            def _sidx(j, c2):
                for g in range(BLK // 16):
                    sl = pl.ds(g * 16, 16)
                    sidxb[j, sl] = jnp.where(etb[j, sl] == cid, dstb[j, sl],
                                             DUMMY)
                return c2
            lax.fori_loop(0, SUP, _sidx, 0)

            def _blk(j, c2):
                pltpu.async_copy(x.at[srcb.at[j]], rows, sem).wait()
                pltpu.sync_copy(rows, acc.at[sidxb.at[j]], add=True)
                return c2
            lax.fori_loop(0, SUP, _blk, 0)
            return carry
        lax.fori_loop(0, n_sup, _sup, 0)
        plsc.subcore_barrier()

        pltpu.sync_copy(acc.at[pl.ds(sid * RPT, RPT)],
                        sp.at[cid, pl.ds(sid * RPT, RPT)])

    return pl.kernel(
        body,
        out_type=jax.ShapeDtypeStruct((NREL, NROW, D), jnp.float32),
        mesh=mesh,
        scratch_types=[
            pltpu.VMEM_SHARED((NROW, D), jnp.float32),
            pltpu.VMEM((SUP, BLK), jnp.int32),
            pltpu.VMEM((SUP, BLK), jnp.int32),
            pltpu.VMEM((SUP, BLK), jnp.int32),
            pltpu.VMEM((SUP, BLK), jnp.int32),
            pltpu.VMEM((BLK, D), jnp.float32),
            pltpu.VMEM((64, D), jnp.float32),
            pltpu.SemaphoreType.DMA,
        ],
    )


def _make_cnt_kernel(n_sup):
    rows_per_tile = n_sup * SUP
    mesh = plsc.VectorSubcoreMesh(core_axis_name="c", subcore_axis_name="s")

    def body(dsth, eth, cnto, acc, dstb, etb, sidxb, ones, zbuf):
        cid = lax.axis_index("c")
        sid = lax.axis_index("s")

        def _fill(i, carry):
            for g in range(D // 16):
                ones[i, pl.ds(g * 16, 16)] = jnp.ones((16,), jnp.float32)
            return carry
        lax.fori_loop(0, BLK, _fill, 0)

        def _fillz(i, carry):
            for g in range(D // 16):
                zbuf[i, pl.ds(g * 16, 16)] = jnp.zeros((16,), jnp.float32)
            return carry
        lax.fori_loop(0, 64, _fillz, 0)

        _zero_acc(zbuf, acc, sid, D)
        plsc.subcore_barrier()

        base0 = sid * rows_per_tile

        def _sup(t, carry):
            base = base0 + t * SUP
            pltpu.sync_copy(dsth.at[pl.ds(base, SUP)], dstb)
            pltpu.sync_copy(eth.at[pl.ds(base, SUP)], etb)

            def _sidx(j, c2):
                for g in range(BLK // 16):
                    sl = pl.ds(g * 16, 16)
                    sidxb[j, sl] = jnp.where(etb[j, sl] == cid, dstb[j, sl],
                                             DUMMY)
                return c2
            lax.fori_loop(0, SUP, _sidx, 0)

            def _blk(j, c2):
                pltpu.sync_copy(ones, acc.at[sidxb.at[j]], add=True)
                return c2
            lax.fori_loop(0, SUP, _blk, 0)
            return carry
        lax.fori_loop(0, n_sup, _sup, 0)
        plsc.subcore_barrier()

        pltpu.sync_copy(acc.at[pl.ds(sid * RPT, RPT)],
                        cnto.at[cid, pl.ds(sid * RPT, RPT)])

    return pl.kernel(
        body,
        out_type=jax.ShapeDtypeStruct((NREL, NROW, D), jnp.float32),
        mesh=mesh,
        scratch_types=[
            pltpu.VMEM_SHARED((NROW, D), jnp.float32),
            pltpu.VMEM((SUP, BLK), jnp.int32),
            pltpu.VMEM((SUP, BLK), jnp.int32),
            pltpu.VMEM((SUP, BLK), jnp.int32),
            pltpu.VMEM((BLK, D), jnp.float32),
            pltpu.VMEM((64, D), jnp.float32),
        ],
    )


_RB = 2000


def _pre_body(cp, wcat, bcat, win, binp, out):
    c = _leaky(jnp.dot(cp[...], wcat[...],
                       preferred_element_type=jnp.float32) + bcat[...])
    out[...] = _leaky(jnp.dot(c, win[...],
                              preferred_element_type=jnp.float32) + binp[...])


def _pre(cat_prop, W_cat, b_cat, W_in, b_in):
    return pl.pallas_call(
        _pre_body,
        grid=(N // _RB,),
        in_specs=[
            pl.BlockSpec((_RB, 11), lambda i: (i, 0)),
            pl.BlockSpec((11, D), lambda i: (0, 0)),
            pl.BlockSpec((1, D), lambda i: (0, 0)),
            pl.BlockSpec((D, D), lambda i: (0, 0)),
            pl.BlockSpec((1, D), lambda i: (0, 0)),
        ],
        out_specs=pl.BlockSpec((_RB, D), lambda i: (i, 0)),
        out_shape=jax.ShapeDtypeStruct((N, D), jnp.float32),
    )(cat_prop, W_cat, b_cat, W_in, b_in)


def _make_comb_body(head):
    def body(x_ref, sp, cp, wroot, wrel, b, *rest):
        if head:
            wo1, bo1, wo2, bo2, out = rest
        else:
            (out,) = rest
        x = x_ref[...]
        o = jnp.dot(x, wroot[...], preferred_element_type=jnp.float32) + b[...]
        for r in range(NREL):
            cnt = cp[r, :, 0]
            inv = 1.0 / jnp.maximum(cnt, 1.0)
            o = o + jnp.dot(sp[r] * inv[:, None], wrel[r],
                            preferred_element_type=jnp.float32)
        if head:
            y = _leaky(jnp.dot(o, wo1[...],
                               preferred_element_type=jnp.float32) + bo1[...])
            out[...] = jnp.dot(y, wo2[...],
                               preferred_element_type=jnp.float32) + bo2[...]
        else:
            out[...] = o
    return body


def _comb_specs():
    return [
        pl.BlockSpec((_RB, D), lambda i: (i, 0)),
        pl.BlockSpec((NREL, _RB, D), lambda i: (0, i, 0)),
        pl.BlockSpec((NREL, _RB, D), lambda i: (0, i, 0)),
        pl.BlockSpec((D, D), lambda i: (0, 0)),
        pl.BlockSpec((NREL, D, D), lambda i: (0, 0, 0)),
        pl.BlockSpec((1, D), lambda i: (0, 0)),
    ]


def _comb1(x, sp, cp, W_root, W_rel, b):
    return pl.pallas_call(
        _make_comb_body(False),
        grid=(N // _RB,),
        in_specs=_comb_specs(),
        out_specs=pl.BlockSpec((_RB, D), lambda i: (i, 0)),
        out_shape=jax.ShapeDtypeStruct((N, D), jnp.float32),
    )(x, sp, cp, W_root, W_rel, b)


def _comb2(x, sp, cp, W_root, W_rel, b, W_o1, b_o1, W_o2, b_o2):
    return pl.pallas_call(
        _make_comb_body(True),
        grid=(N // _RB,),
        in_specs=_comb_specs() + [
            pl.BlockSpec((D, D), lambda i: (0, 0)),
            pl.BlockSpec((1, D), lambda i: (0, 0)),
            pl.BlockSpec((D, 2), lambda i: (0, 0)),
            pl.BlockSpec((1, 2), lambda i: (0, 0)),
        ],
        out_specs=pl.BlockSpec((_RB, 2), lambda i: (i, 0)),
        out_shape=jax.ShapeDtypeStruct((N, 2), jnp.float32),
    )(x, sp, cp, W_root, W_rel, b, W_o1, b_o1, W_o2, b_o2)


def kernel(des, tweet, num_prop, cat_prop, edge_index, edge_type,
           W_cat, b_cat, W_in, b_in, W_rel, W_root, b_rgcn,
           W_o1, b_o1, W_o2, b_o2):
    del des, tweet, num_prop
    E = edge_index.shape[1]
    src = edge_index[0].astype(jnp.int32)
    dst = edge_index[1].astype(jnp.int32)
    et = edge_type.astype(jnp.int32)

    chunk = NSUB * SUP * BLK
    epad = (-E) % chunk
    if epad:
        src = jnp.concatenate([src, jnp.zeros((epad,), jnp.int32)])
        dst = jnp.concatenate([dst, jnp.full((epad,), DUMMY, jnp.int32)])
        et = jnp.concatenate([et, jnp.zeros((epad,), jnp.int32)])
    src2 = src.reshape(-1, BLK)
    dst2 = dst.reshape(-1, BLK)
    et2 = et.reshape(-1, BLK)
    n_sup = src2.shape[0] // (NSUB * SUP)

    b_cat2 = b_cat.reshape(1, D)
    b_in2 = b_in.reshape(1, D)
    b_rgcn2 = b_rgcn.reshape(1, D)
    b_o12 = b_o1.reshape(1, D)
    b_o22 = b_o2.reshape(1, 2)

    cnt = _make_cnt_kernel(n_sup)(dst2, et2)

    x0 = _pre(cat_prop, W_cat, b_cat2, W_in, b_in2)
    sp1 = _make_edge_kernel(n_sup)(x0, src2, dst2, et2)
    x1 = _comb1(x0, sp1, cnt, W_root, W_rel, b_rgcn2)
    sp2 = _make_edge_kernel(n_sup)(x1, src2, dst2, et2)
    return _comb2(x1, sp2, cnt, W_root, W_rel, b_rgcn2,
                  W_o1, b_o12, W_o2, b_o22)


# speedup vs baseline: 3.4435x; 1.0471x over previous
import jax
import jax.numpy as jnp
from jax import lax
from jax.experimental import pallas as pl
from jax.experimental.pallas import tpu as pltpu
from jax.experimental.pallas import tpu_sc as plsc

N = 10000
D = 128
NREL = 2
NROW = 10112
NCORE = 2
NSUB = 16
BLK = 128
SUP = 16
DUMMY = N
RPT = NROW // NSUB


def _leaky(x):
    return jnp.where(x >= 0, x, 0.01 * x)


def _zero_acc(zbuf, acc, sid, width):
    del width
    zr = zbuf.shape[0]
    base = sid * RPT
    for k in range(RPT // zr):
        pltpu.sync_copy(zbuf, acc.at[pl.ds(base + k * zr, zr)])
    rem = RPT % zr
    if rem:
        pltpu.sync_copy(zbuf.at[pl.ds(0, rem)],
                        acc.at[pl.ds(base + (RPT // zr) * zr, rem)])


def _make_edge_kernel(n_sup):
    rows_per_tile = n_sup * SUP
    mesh = plsc.VectorSubcoreMesh(core_axis_name="c", subcore_axis_name="s")

    def body(x, srch, dsth, eth, sp, acc, srcb, dstb, etb, sidxb,
             rows_a, rows_b, zbuf, sem_a, sem_b):
        cid = lax.axis_index("c")
        sid = lax.axis_index("s")

        def _fill(i, carry):
            for g in range(D // 16):
                zbuf[i, pl.ds(g * 16, 16)] = jnp.zeros((16,), jnp.float32)
            return carry
        lax.fori_loop(0, 32, _fill, 0)

        _zero_acc(zbuf, acc, sid, D)
        plsc.subcore_barrier()

        base0 = sid * rows_per_tile
        bufs = (rows_a, rows_b)
        sems = (sem_a, sem_b)

        def _sup(t, carry):
            base = base0 + t * SUP
            pltpu.sync_copy(srch.at[pl.ds(base, SUP)], srcb)
            pltpu.sync_copy(dsth.at[pl.ds(base, SUP)], dstb)
            pltpu.sync_copy(eth.at[pl.ds(base, SUP)], etb)

            descs = [pltpu.async_copy(x.at[srcb.at[0]], bufs[0], sems[0])]

            def _sidx(j, c2):
                for g in range(BLK // 16):
                    sl = pl.ds(g * 16, 16)
                    sidxb[j, sl] = jnp.where(etb[j, sl] == cid, dstb[j, sl],
                                             DUMMY)
                return c2
            lax.fori_loop(0, SUP, _sidx, 0)

            for j in range(SUP):
                descs[j].wait()
                if j + 1 < SUP:
                    descs.append(pltpu.async_copy(
                        x.at[srcb.at[j + 1]], bufs[(j + 1) % 2],
                        sems[(j + 1) % 2]))
                pltpu.sync_copy(bufs[j % 2], acc.at[sidxb.at[j]], add=True)
            return carry
        lax.fori_loop(0, n_sup, _sup, 0)
        plsc.subcore_barrier()

        pltpu.sync_copy(acc.at[pl.ds(sid * RPT, RPT)],
                        sp.at[cid, pl.ds(sid * RPT, RPT)])

    return pl.kernel(
        body,
        out_type=jax.ShapeDtypeStruct((NREL, NROW, D), jnp.float32),
        mesh=mesh,
        scratch_types=[
            pltpu.VMEM_SHARED((NROW, D), jnp.float32),
            pltpu.VMEM((SUP, BLK), jnp.int32),
            pltpu.VMEM((SUP, BLK), jnp.int32),
            pltpu.VMEM((SUP, BLK), jnp.int32),
            pltpu.VMEM((SUP, BLK), jnp.int32),
            pltpu.VMEM((BLK, D), jnp.float32),
            pltpu.VMEM((BLK, D), jnp.float32),
            pltpu.VMEM((32, D), jnp.float32),
            pltpu.SemaphoreType.DMA,
            pltpu.SemaphoreType.DMA,
        ],
    )


def _make_cnt_kernel(n_sup):
    rows_per_tile = n_sup * SUP
    mesh = plsc.VectorSubcoreMesh(core_axis_name="c", subcore_axis_name="s")

    def body(dsth, eth, cnto, acc, dstb, etb, sidxb, ones, zbuf):
        cid = lax.axis_index("c")
        sid = lax.axis_index("s")

        def _fill(i, carry):
            for g in range(D // 16):
                ones[i, pl.ds(g * 16, 16)] = jnp.ones((16,), jnp.float32)
            return carry
        lax.fori_loop(0, BLK, _fill, 0)

        def _fillz(i, carry):
            for g in range(D // 16):
                zbuf[i, pl.ds(g * 16, 16)] = jnp.zeros((16,), jnp.float32)
            return carry
        lax.fori_loop(0, 64, _fillz, 0)

        _zero_acc(zbuf, acc, sid, D)
        plsc.subcore_barrier()

        base0 = sid * rows_per_tile

        def _sup(t, carry):
            base = base0 + t * SUP
            pltpu.sync_copy(dsth.at[pl.ds(base, SUP)], dstb)
            pltpu.sync_copy(eth.at[pl.ds(base, SUP)], etb)

            def _sidx(j, c2):
                for g in range(BLK // 16):
                    sl = pl.ds(g * 16, 16)
                    sidxb[j, sl] = jnp.where(etb[j, sl] == cid, dstb[j, sl],
                                             DUMMY)
                return c2
            lax.fori_loop(0, SUP, _sidx, 0)

            def _blk(j, c2):
                pltpu.sync_copy(ones, acc.at[sidxb.at[j]], add=True)
                return c2
            lax.fori_loop(0, SUP, _blk, 0)
            return carry
        lax.fori_loop(0, n_sup, _sup, 0)
        plsc.subcore_barrier()

        pltpu.sync_copy(acc.at[pl.ds(sid * RPT, RPT)],
                        cnto.at[cid, pl.ds(sid * RPT, RPT)])

    return pl.kernel(
        body,
        out_type=jax.ShapeDtypeStruct((NREL, NROW, D), jnp.float32),
        mesh=mesh,
        scratch_types=[
            pltpu.VMEM_SHARED((NROW, D), jnp.float32),
            pltpu.VMEM((SUP, BLK), jnp.int32),
            pltpu.VMEM((SUP, BLK), jnp.int32),
            pltpu.VMEM((SUP, BLK), jnp.int32),
            pltpu.VMEM((BLK, D), jnp.float32),
            pltpu.VMEM((64, D), jnp.float32),
        ],
    )


_RB = 2000


def _pre_body(cp, wcat, bcat, win, binp, out):
    c = _leaky(jnp.dot(cp[...], wcat[...],
                       preferred_element_type=jnp.float32) + bcat[...])
    out[...] = _leaky(jnp.dot(c, win[...],
                              preferred_element_type=jnp.float32) + binp[...])


def _pre(cat_prop, W_cat, b_cat, W_in, b_in):
    return pl.pallas_call(
        _pre_body,
        grid=(N // _RB,),
        in_specs=[
            pl.BlockSpec((_RB, 11), lambda i: (i, 0)),
            pl.BlockSpec((11, D), lambda i: (0, 0)),
            pl.BlockSpec((1, D), lambda i: (0, 0)),
            pl.BlockSpec((D, D), lambda i: (0, 0)),
            pl.BlockSpec((1, D), lambda i: (0, 0)),
        ],
        out_specs=pl.BlockSpec((_RB, D), lambda i: (i, 0)),
        out_shape=jax.ShapeDtypeStruct((N, D), jnp.float32),
    )(cat_prop, W_cat, b_cat, W_in, b_in)


def _make_comb_body(head):
    def body(x_ref, sp, cp, wroot, wrel, b, *rest):
        if head:
            wo1, bo1, wo2, bo2, out = rest
        else:
            (out,) = rest
        x = x_ref[...]
        o = jnp.dot(x, wroot[...], preferred_element_type=jnp.float32) + b[...]
        for r in range(NREL):
            cnt = cp[r, :, 0]
            inv = 1.0 / jnp.maximum(cnt, 1.0)
            o = o + jnp.dot(sp[r] * inv[:, None], wrel[r],
                            preferred_element_type=jnp.float32)
        if head:
            y = _leaky(jnp.dot(o, wo1[...],
                               preferred_element_type=jnp.float32) + bo1[...])
            out[...] = jnp.dot(y, wo2[...],
                               preferred_element_type=jnp.float32) + bo2[...]
        else:
            out[...] = o
    return body


def _comb_specs():
    return [
        pl.BlockSpec((_RB, D), lambda i: (i, 0)),
        pl.BlockSpec((NREL, _RB, D), lambda i: (0, i, 0)),
        pl.BlockSpec((NREL, _RB, D), lambda i: (0, i, 0)),
        pl.BlockSpec((D, D), lambda i: (0, 0)),
        pl.BlockSpec((NREL, D, D), lambda i: (0, 0, 0)),
        pl.BlockSpec((1, D), lambda i: (0, 0)),
    ]


def _comb1(x, sp, cp, W_root, W_rel, b):
    return pl.pallas_call(
        _make_comb_body(False),
        grid=(N // _RB,),
        in_specs=_comb_specs(),
        out_specs=pl.BlockSpec((_RB, D), lambda i: (i, 0)),
        out_shape=jax.ShapeDtypeStruct((N, D), jnp.float32),
    )(x, sp, cp, W_root, W_rel, b)


def _comb2(x, sp, cp, W_root, W_rel, b, W_o1, b_o1, W_o2, b_o2):
    return pl.pallas_call(
        _make_comb_body(True),
        grid=(N // _RB,),
        in_specs=_comb_specs() + [
            pl.BlockSpec((D, D), lambda i: (0, 0)),
            pl.BlockSpec((1, D), lambda i: (0, 0)),
            pl.BlockSpec((D, 2), lambda i: (0, 0)),
            pl.BlockSpec((1, 2), lambda i: (0, 0)),
        ],
        out_specs=pl.BlockSpec((_RB, 2), lambda i: (i, 0)),
        out_shape=jax.ShapeDtypeStruct((N, 2), jnp.float32),
    )(x, sp, cp, W_root, W_rel, b, W_o1, b_o1, W_o2, b_o2)


def kernel(des, tweet, num_prop, cat_prop, edge_index, edge_type,
           W_cat, b_cat, W_in, b_in, W_rel, W_root, b_rgcn,
           W_o1, b_o1, W_o2, b_o2):
    del des, tweet, num_prop
    E = edge_index.shape[1]
    src = edge_index[0].astype(jnp.int32)
    dst = edge_index[1].astype(jnp.int32)
    et = edge_type.astype(jnp.int32)

    chunk = NSUB * SUP * BLK
    epad = (-E) % chunk
    if epad:
        src = jnp.concatenate([src, jnp.zeros((epad,), jnp.int32)])
        dst = jnp.concatenate([dst, jnp.full((epad,), DUMMY, jnp.int32)])
        et = jnp.concatenate([et, jnp.zeros((epad,), jnp.int32)])
    src2 = src.reshape(-1, BLK)
    dst2 = dst.reshape(-1, BLK)
    et2 = et.reshape(-1, BLK)
    n_sup = src2.shape[0] // (NSUB * SUP)

    b_cat2 = b_cat.reshape(1, D)
    b_in2 = b_in.reshape(1, D)
    b_rgcn2 = b_rgcn.reshape(1, D)
    b_o12 = b_o1.reshape(1, D)
    b_o22 = b_o2.reshape(1, 2)

    cnt = _make_cnt_kernel(n_sup)(dst2, et2)

    x0 = _pre(cat_prop, W_cat, b_cat2, W_in, b_in2)
    sp1 = _make_edge_kernel(n_sup)(x0, src2, dst2, et2)
    x1 = _comb1(x0, sp1, cnt, W_root, W_rel, b_rgcn2)
    sp2 = _make_edge_kernel(n_sup)(x1, src2, dst2, et2)
    return _comb2(x1, sp2, cnt, W_root, W_rel, b_rgcn2,
                  W_o1, b_o12, W_o2, b_o22)


# speedup vs baseline: 7.2879x; 2.1164x over previous
import jax
import jax.numpy as jnp
from jax import lax
from jax.experimental import pallas as pl
from jax.experimental.pallas import tpu as pltpu
from jax.experimental.pallas import tpu_sc as plsc

N = 10000
D = 128
H = 64
NREL = 2
NROW = 10112
NCORE = 2
NSUB = 16
BLK = 128
SUP = 16
DUMMY = N
NDEEP = 3


def _leaky(x):
    return jnp.where(x >= 0, x, 0.01 * x)


def _zero_acc(zbuf, acc, sid, rpt):
    zr = zbuf.shape[0]
    base = sid * rpt
    for k in range(rpt // zr):
        pltpu.sync_copy(zbuf, acc.at[pl.ds(base + k * zr, zr)])
    rem = rpt % zr
    if rem:
        pltpu.sync_copy(zbuf.at[pl.ds(0, rem)],
                        acc.at[pl.ds(base + (rpt // zr) * zr, rem)])


_SC_PARAMS = pltpu.CompilerParams(use_tc_tiling_on_sc=False)


def _make_edge_kernel(n_sup):
    rows_per_tile = n_sup * SUP
    rpt_acc = NREL * NROW // NSUB
    rpt_out = NROW // NSUB
    mesh = plsc.VectorSubcoreMesh(core_axis_name="c", subcore_axis_name="s")

    def body(xh, srch, dsth, eth, sp, acc, srcb, dstb, etb, sidxb,
             rows_a, rows_b, rows_c, zbuf, sem_a, sem_b, sem_c):
        cid = lax.axis_index("c")
        sid = lax.axis_index("s")

        def _fill(i, carry):
            for g in range(H // 16):
                zbuf[i, pl.ds(g * 16, 16)] = jnp.zeros((16,), jnp.float32)
            return carry
        lax.fori_loop(0, 32, _fill, 0)

        _zero_acc(zbuf, acc, sid, rpt_acc)
        plsc.subcore_barrier()

        base0 = sid * rows_per_tile
        bufs = (rows_a, rows_b, rows_c)
        sems = (sem_a, sem_b, sem_c)
        src_off = cid * N

        def _sup(t, carry):
            base = base0 + t * SUP
            pltpu.sync_copy(srch.at[pl.ds(base, SUP)], srcb)
            pltpu.sync_copy(dsth.at[pl.ds(base, SUP)], dstb)
            pltpu.sync_copy(eth.at[pl.ds(base, SUP)], etb)

            def _gidx(j, c2):
                for g in range(BLK // 16):
                    sl = pl.ds(g * 16, 16)
                    srcb[j, sl] = srcb[j, sl] + src_off
                    sidxb[j, sl] = etb[j, sl] * NROW + dstb[j, sl]
                return c2
            lax.fori_loop(0, 2, _gidx, 0)

            descs = [pltpu.async_copy(xh.at[srcb.at[0]], bufs[0], sems[0]),
                     pltpu.async_copy(xh.at[srcb.at[1]], bufs[1], sems[1])]

            def _gidx2(j, c2):
                return _gidx(j, c2)
            lax.fori_loop(2, SUP, _gidx2, 0)

            for j in range(SUP):
                descs[j].wait()
                if j + 2 < SUP:
                    descs.append(pltpu.async_copy(
                        xh.at[srcb.at[j + 2]], bufs[(j + 2) % NDEEP],
                        sems[(j + 2) % NDEEP]))
                pltpu.sync_copy(bufs[j % NDEEP], acc.at[sidxb.at[j]],
                                add=True)
            return carry
        lax.fori_loop(0, n_sup, _sup, 0)
        plsc.subcore_barrier()

        for r in range(NREL):
            pltpu.sync_copy(
                acc.at[pl.ds(r * NROW + sid * rpt_out, rpt_out)],
                sp.at[cid, r, pl.ds(sid * rpt_out, rpt_out)])

    return pl.kernel(
        body,
        out_type=jax.ShapeDtypeStruct((NCORE, NREL, NROW, H), jnp.float32),
        mesh=mesh,
        compiler_params=_SC_PARAMS,
        scratch_types=[
            pltpu.VMEM_SHARED((NREL * NROW, H), jnp.float32),
            pltpu.VMEM((SUP, BLK), jnp.int32),
            pltpu.VMEM((SUP, BLK), jnp.int32),
            pltpu.VMEM((SUP, BLK), jnp.int32),
            pltpu.VMEM((SUP, BLK), jnp.int32),
            pltpu.VMEM((BLK, H), jnp.float32),
            pltpu.VMEM((BLK, H), jnp.float32),
            pltpu.VMEM((BLK, H), jnp.float32),
            pltpu.VMEM((32, H), jnp.float32),
            pltpu.SemaphoreType.DMA,
            pltpu.SemaphoreType.DMA,
            pltpu.SemaphoreType.DMA,
        ],
    )


def _make_cnt_kernel(n_sup):
    rows_per_tile = n_sup * SUP
    rpt_out = NROW // NSUB
    mesh = plsc.VectorSubcoreMesh(core_axis_name="c", subcore_axis_name="s")

    def body(dsth, eth, cnto, acc, dstb, etb, sidxb, ones, zbuf):
        cid = lax.axis_index("c")
        sid = lax.axis_index("s")

        def _fill(i, carry):
            ones[i, pl.ds(0, 16)] = jnp.ones((16,), jnp.float32)
            return carry
        lax.fori_loop(0, BLK, _fill, 0)

        def _fillz(i, carry):
            zbuf[i, pl.ds(0, 16)] = jnp.zeros((16,), jnp.float32)
            return carry
        lax.fori_loop(0, 64, _fillz, 0)

        _zero_acc(zbuf, acc, sid, rpt_out)
        plsc.subcore_barrier()

        base0 = sid * rows_per_tile

        def _sup(t, carry):
            base = base0 + t * SUP
            pltpu.sync_copy(dsth.at[pl.ds(base, SUP)], dstb)
            pltpu.sync_copy(eth.at[pl.ds(base, SUP)], etb)

            def _sidx(j, c2):
                for g in range(BLK // 16):
                    sl = pl.ds(g * 16, 16)
                    sidxb[j, sl] = jnp.where(etb[j, sl] == cid, dstb[j, sl],
                                             DUMMY)
                return c2
            lax.fori_loop(0, SUP, _sidx, 0)

            def _blk(j, c2):
                pltpu.sync_copy(ones, acc.at[sidxb.at[j]], add=True)
                return c2
            lax.fori_loop(0, SUP, _blk, 0)
            return carry
        lax.fori_loop(0, n_sup, _sup, 0)
        plsc.subcore_barrier()

        pltpu.sync_copy(acc.at[pl.ds(sid * rpt_out, rpt_out)],
                        cnto.at[cid, pl.ds(sid * rpt_out, rpt_out)])

    return pl.kernel(
        body,
        out_type=jax.ShapeDtypeStruct((NREL, NROW, 16), jnp.float32),
        mesh=mesh,
        compiler_params=_SC_PARAMS,
        scratch_types=[
            pltpu.VMEM_SHARED((NROW, 16), jnp.float32),
            pltpu.VMEM((SUP, BLK), jnp.int32),
            pltpu.VMEM((SUP, BLK), jnp.int32),
            pltpu.VMEM((SUP, BLK), jnp.int32),
            pltpu.VMEM((BLK, 16), jnp.float32),
            pltpu.VMEM((64, 16), jnp.float32),
        ],
    )


_RB = 2000


def _pre_body(cp, wcat, bcat, win, binp, out):
    c = _leaky(jnp.dot(cp[...], wcat[...],
                       preferred_element_type=jnp.float32) + bcat[...])
    x = _leaky(jnp.dot(c, win[...],
                       preferred_element_type=jnp.float32) + binp[...])
    out[0] = x[:, :H]
    out[1] = x[:, H:]


def _pre(cat_prop, W_cat, b_cat, W_in, b_in):
    return pl.pallas_call(
        _pre_body,
        grid=(N // _RB,),
        in_specs=[
            pl.BlockSpec((_RB, 11), lambda i: (i, 0)),
            pl.BlockSpec((11, D), lambda i: (0, 0)),
            pl.BlockSpec((1, D), lambda i: (0, 0)),
            pl.BlockSpec((D, D), lambda i: (0, 0)),
            pl.BlockSpec((1, D), lambda i: (0, 0)),
        ],
        out_specs=pl.BlockSpec((2, _RB, H), lambda i: (0, i, 0)),
        out_shape=jax.ShapeDtypeStruct((2, N, H), jnp.float32),
    )(cat_prop, W_cat, b_cat, W_in, b_in)


def _make_comb_body(head):
    def body(xs, sp, cp, wroot, wrel, b, *rest):
        if head:
            wo1, bo1, wo2, bo2, out = rest
        else:
            (out,) = rest
        x = jnp.concatenate([xs[0], xs[1]], axis=1)
        o = jnp.dot(x, wroot[...], preferred_element_type=jnp.float32) + b[...]
        for r in range(NREL):
            s = jnp.concatenate([sp[0, r], sp[1, r]], axis=1)
            cnt = cp[r, :, 0]
            inv = 1.0 / jnp.maximum(cnt, 1.0)
            o = o + jnp.dot(s * inv[:, None], wrel[r],
                            preferred_element_type=jnp.float32)
        if head:
            y = _leaky(jnp.dot(o, wo1[...],
                               preferred_element_type=jnp.float32) + bo1[...])
            out[...] = jnp.dot(y, wo2[...],
                               preferred_element_type=jnp.float32) + bo2[...]
        else:
            out[0] = o[:, :H]
            out[1] = o[:, H:]
    return body


def _comb_specs():
    return [
        pl.BlockSpec((2, _RB, H), lambda i: (0, i, 0)),
        pl.BlockSpec((NCORE, NREL, _RB, H), lambda i: (0, 0, i, 0)),
        pl.BlockSpec((NREL, _RB, 16), lambda i: (0, i, 0)),
        pl.BlockSpec((D, D), lambda i: (0, 0)),
        pl.BlockSpec((NREL, D, D), lambda i: (0, 0, 0)),
        pl.BlockSpec((1, D), lambda i: (0, 0)),
    ]


def _comb1(xs, sp, cp, W_root, W_rel, b):
    return pl.pallas_call(
        _make_comb_body(False),
        grid=(N // _RB,),
        in_specs=_comb_specs(),
        out_specs=pl.BlockSpec((2, _RB, H), lambda i: (0, i, 0)),
        out_shape=jax.ShapeDtypeStruct((2, N, H), jnp.float32),
    )(xs, sp, cp, W_root, W_rel, b)


def _comb2(xs, sp, cp, W_root, W_rel, b, W_o1, b_o1, W_o2, b_o2):
    return pl.pallas_call(
        _make_comb_body(True),
        grid=(N // _RB,),
        in_specs=_comb_specs() + [
            pl.BlockSpec((D, D), lambda i: (0, 0)),
            pl.BlockSpec((1, D), lambda i: (0, 0)),
            pl.BlockSpec((D, 2), lambda i: (0, 0)),
            pl.BlockSpec((1, 2), lambda i: (0, 0)),
        ],
        out_specs=pl.BlockSpec((_RB, 2), lambda i: (i, 0)),
        out_shape=jax.ShapeDtypeStruct((N, 2), jnp.float32),
    )(xs, sp, cp, W_root, W_rel, b, W_o1, b_o1, W_o2, b_o2)


def kernel(des, tweet, num_prop, cat_prop, edge_index, edge_type,
           W_cat, b_cat, W_in, b_in, W_rel, W_root, b_rgcn,
           W_o1, b_o1, W_o2, b_o2):
    del des, tweet, num_prop
    E = edge_index.shape[1]
    src = edge_index[0].astype(jnp.int32)
    dst = edge_index[1].astype(jnp.int32)
    et = edge_type.astype(jnp.int32)

    chunk = NSUB * SUP * BLK
    epad = (-E) % chunk
    if epad:
        src = jnp.concatenate([src, jnp.zeros((epad,), jnp.int32)])
        dst = jnp.concatenate([dst, jnp.full((epad,), DUMMY, jnp.int32)])
        et = jnp.concatenate([et, jnp.zeros((epad,), jnp.int32)])
    src2 = src.reshape(-1, BLK)
    dst2 = dst.reshape(-1, BLK)
    et2 = et.reshape(-1, BLK)
    n_sup = src2.shape[0] // (NSUB * SUP)

    b_cat2 = b_cat.reshape(1, D)
    b_in2 = b_in.reshape(1, D)
    b_rgcn2 = b_rgcn.reshape(1, D)
    b_o12 = b_o1.reshape(1, D)
    b_o22 = b_o2.reshape(1, 2)

    cnt = _make_cnt_kernel(n_sup)(dst2, et2)
    edge = _make_edge_kernel(n_sup)

    xs0 = _pre(cat_prop, W_cat, b_cat2, W_in, b_in2)
    sp1 = edge(xs0.reshape(2 * N, H), src2, dst2, et2)
    xs1 = _comb1(xs0, sp1, cnt, W_root, W_rel, b_rgcn2)
    sp2 = edge(xs1.reshape(2 * N, H), src2, dst2, et2)
    return _comb2(xs1, sp2, cnt, W_root, W_rel, b_rgcn2,
                  W_o1, b_o12, W_o2, b_o22)


# speedup vs baseline: 8.9009x; 1.2213x over previous
import jax
import jax.numpy as jnp
from jax import lax
from jax.experimental import pallas as pl
from jax.experimental.pallas import tpu as pltpu
from jax.experimental.pallas import tpu_sc as plsc

N = 10000
D = 128
H = 64
NREL = 2
NROW = 10112
NCORE = 2
NSUB = 16
BLK = 128
SUP = 16
DUMMY = N
NDEEP = 4


def _leaky(x):
    return jnp.where(x >= 0, x, 0.01 * x)


def _zero_acc(zbuf, acc, sid, rpt):
    zr = zbuf.shape[0]
    base = sid * rpt
    for k in range(rpt // zr):
        pltpu.sync_copy(zbuf, acc.at[pl.ds(base + k * zr, zr)])
    rem = rpt % zr
    if rem:
        pltpu.sync_copy(zbuf.at[pl.ds(0, rem)],
                        acc.at[pl.ds(base + (rpt // zr) * zr, rem)])


_SC_PARAMS = pltpu.CompilerParams(use_tc_tiling_on_sc=False)


def _make_edge_kernel(n_sup):
    rows_per_tile = n_sup * SUP
    rpt_acc = NREL * NROW // NSUB
    rpt_out = NROW // NSUB
    mesh = plsc.VectorSubcoreMesh(core_axis_name="c", subcore_axis_name="s")

    def body(xh, srch, dsth, eth, sp, acc, srcb, dstb, etb, sidxb,
             rows_a, rows_b, rows_c, rows_d, zbuf, sem_a, sem_b, sem_c, sem_d):
        cid = lax.axis_index("c")
        sid = lax.axis_index("s")

        def _fill(i, carry):
            for g in range(H // 16):
                zbuf[i, pl.ds(g * 16, 16)] = jnp.zeros((16,), jnp.float32)
            return carry
        lax.fori_loop(0, 32, _fill, 0)

        _zero_acc(zbuf, acc, sid, rpt_acc)
        plsc.subcore_barrier()

        base0 = sid * rows_per_tile
        bufs = (rows_a, rows_b, rows_c, rows_d)
        sems = (sem_a, sem_b, sem_c, sem_d)
        src_off = cid * N

        def _sup(t, carry):
            base = base0 + t * SUP
            pltpu.sync_copy(srch.at[pl.ds(base, SUP)], srcb)
            pltpu.sync_copy(dsth.at[pl.ds(base, SUP)], dstb)
            pltpu.sync_copy(eth.at[pl.ds(base, SUP)], etb)

            nfly = NDEEP - 1
            def _gidx(j, c2):
                for g in range(BLK // 16):
                    sl = pl.ds(g * 16, 16)
                    srcb[j, sl] = srcb[j, sl] + src_off
                    sidxb[j, sl] = etb[j, sl] * NROW + dstb[j, sl]
                return c2
            lax.fori_loop(0, nfly, _gidx, 0)

            descs = [pltpu.async_copy(xh.at[srcb.at[k]], bufs[k], sems[k])
                     for k in range(nfly)]

            def _gidx2(j, c2):
                return _gidx(j, c2)
            lax.fori_loop(nfly, SUP, _gidx2, 0)

            for j in range(SUP):
                descs[j].wait()
                if j + nfly < SUP:
                    descs.append(pltpu.async_copy(
                        xh.at[srcb.at[j + nfly]], bufs[(j + nfly) % NDEEP],
                        sems[(j + nfly) % NDEEP]))
                pltpu.sync_copy(bufs[j % NDEEP], acc.at[sidxb.at[j]],
                                add=True)
            return carry
        lax.fori_loop(0, n_sup, _sup, 0)
        plsc.subcore_barrier()

        for r in range(NREL):
            pltpu.sync_copy(
                acc.at[pl.ds(r * NROW + sid * rpt_out, rpt_out)],
                sp.at[cid, r, pl.ds(sid * rpt_out, rpt_out)])

    return pl.kernel(
        body,
        out_type=jax.ShapeDtypeStruct((NCORE, NREL, NROW, H), jnp.float32),
        mesh=mesh,
        compiler_params=_SC_PARAMS,
        scratch_types=[
            pltpu.VMEM_SHARED((NREL * NROW, H), jnp.float32),
            pltpu.VMEM((SUP, BLK), jnp.int32),
            pltpu.VMEM((SUP, BLK), jnp.int32),
            pltpu.VMEM((SUP, BLK), jnp.int32),
            pltpu.VMEM((SUP, BLK), jnp.int32),
            pltpu.VMEM((BLK, H), jnp.float32),
            pltpu.VMEM((BLK, H), jnp.float32),
            pltpu.VMEM((BLK, H), jnp.float32),
            pltpu.VMEM((BLK, H), jnp.float32),
            pltpu.VMEM((32, H), jnp.float32),
            pltpu.SemaphoreType.DMA,
            pltpu.SemaphoreType.DMA,
            pltpu.SemaphoreType.DMA,
            pltpu.SemaphoreType.DMA,
        ],
    )


def _make_cnt_kernel(n_sup):
    rows_per_tile = n_sup * SUP // NCORE
    rpt_acc = NREL * NROW // NSUB
    rpt_out = NROW // NSUB
    mesh = plsc.VectorSubcoreMesh(core_axis_name="c", subcore_axis_name="s")

    def body(dsth, eth, cnto, acc, dstb, etb, sidxb, ones, zbuf):
        cid = lax.axis_index("c")
        sid = lax.axis_index("s")

        def _fill(i, carry):
            ones[i, pl.ds(0, 16)] = jnp.ones((16,), jnp.float32)
            return carry
        lax.fori_loop(0, BLK, _fill, 0)

        def _fillz(i, carry):
            zbuf[i, pl.ds(0, 16)] = jnp.zeros((16,), jnp.float32)
            return carry
        lax.fori_loop(0, 64, _fillz, 0)

        _zero_acc(zbuf, acc, sid, rpt_acc)
        plsc.subcore_barrier()

        base0 = (cid * NSUB + sid) * rows_per_tile

        def _sup(t, carry):
            base = base0 + t * SUP
            pltpu.sync_copy(dsth.at[pl.ds(base, SUP)], dstb)
            pltpu.sync_copy(eth.at[pl.ds(base, SUP)], etb)

            def _sidx(j, c2):
                for g in range(BLK // 16):
                    sl = pl.ds(g * 16, 16)
                    sidxb[j, sl] = etb[j, sl] * NROW + dstb[j, sl]
                return c2
            lax.fori_loop(0, SUP, _sidx, 0)

            def _blk(j, c2):
                pltpu.sync_copy(ones, acc.at[sidxb.at[j]], add=True)
                return c2
            lax.fori_loop(0, SUP, _blk, 0)
            return carry
        lax.fori_loop(0, n_sup // NCORE, _sup, 0)
        plsc.subcore_barrier()

        for r in range(NREL):
            pltpu.sync_copy(
                acc.at[pl.ds(r * NROW + sid * rpt_out, rpt_out)],
                cnto.at[cid, r, pl.ds(sid * rpt_out, rpt_out)])

    return pl.kernel(
        body,
        out_type=jax.ShapeDtypeStruct((NCORE, NREL, NROW, 16), jnp.float32),
        mesh=mesh,
        compiler_params=_SC_PARAMS,
        scratch_types=[
            pltpu.VMEM_SHARED((NREL * NROW, 16), jnp.float32),
            pltpu.VMEM((SUP, BLK), jnp.int32),
            pltpu.VMEM((SUP, BLK), jnp.int32),
            pltpu.VMEM((SUP, BLK), jnp.int32),
            pltpu.VMEM((BLK, 16), jnp.float32),
            pltpu.VMEM((64, 16), jnp.float32),
        ],
    )


_RB = 2000


def _pre_body(cp, wcat, bcat, win, binp, out):
    c = _leaky(jnp.dot(cp[...], wcat[...],
                       preferred_element_type=jnp.float32) + bcat[...])
    x = _leaky(jnp.dot(c, win[...],
                       preferred_element_type=jnp.float32) + binp[...])
    out[0] = x[:, :H]
    out[1] = x[:, H:]


def _pre(cat_prop, W_cat, b_cat, W_in, b_in):
    return pl.pallas_call(
        _pre_body,
        grid=(N // _RB,),
        in_specs=[
            pl.BlockSpec((_RB, 11), lambda i: (i, 0)),
            pl.BlockSpec((11, D), lambda i: (0, 0)),
            pl.BlockSpec((1, D), lambda i: (0, 0)),
            pl.BlockSpec((D, D), lambda i: (0, 0)),
            pl.BlockSpec((1, D), lambda i: (0, 0)),
        ],
        out_specs=pl.BlockSpec((2, _RB, H), lambda i: (0, i, 0)),
        out_shape=jax.ShapeDtypeStruct((2, N, H), jnp.float32),
    )(cat_prop, W_cat, b_cat, W_in, b_in)


def _make_comb_body(head):
    def body(xs, sp, cp, wroot, wrel, b, *rest):
        if head:
            wo1, bo1, wo2, bo2, out = rest
        else:
            (out,) = rest
        x = jnp.concatenate([xs[0], xs[1]], axis=1)
        o = jnp.dot(x, wroot[...], preferred_element_type=jnp.float32) + b[...]
        for r in range(NREL):
            s = jnp.concatenate([sp[0, r], sp[1, r]], axis=1)
            cnt = cp[0, r, :, 0] + cp[1, r, :, 0]
            inv = 1.0 / jnp.maximum(cnt, 1.0)
            o = o + jnp.dot(s * inv[:, None], wrel[r],
                            preferred_element_type=jnp.float32)
        if head:
            y = _leaky(jnp.dot(o, wo1[...],
                               preferred_element_type=jnp.float32) + bo1[...])
            out[...] = jnp.dot(y, wo2[...],
                               preferred_element_type=jnp.float32) + bo2[...]
        else:
            out[0] = o[:, :H]
            out[1] = o[:, H:]
    return body


def _comb_specs():
    return [
        pl.BlockSpec((2, _RB, H), lambda i: (0, i, 0)),
        pl.BlockSpec((NCORE, NREL, _RB, H), lambda i: (0, 0, i, 0)),
        pl.BlockSpec((NCORE, NREL, _RB, 16), lambda i: (0, 0, i, 0)),
        pl.BlockSpec((D, D), lambda i: (0, 0)),
        pl.BlockSpec((NREL, D, D), lambda i: (0, 0, 0)),
        pl.BlockSpec((1, D), lambda i: (0, 0)),
    ]


def _comb1(xs, sp, cp, W_root, W_rel, b):
    return pl.pallas_call(
        _make_comb_body(False),
        grid=(N // _RB,),
        in_specs=_comb_specs(),
        out_specs=pl.BlockSpec((2, _RB, H), lambda i: (0, i, 0)),
        out_shape=jax.ShapeDtypeStruct((2, N, H), jnp.float32),
    )(xs, sp, cp, W_root, W_rel, b)


def _comb2(xs, sp, cp, W_root, W_rel, b, W_o1, b_o1, W_o2, b_o2):
    return pl.pallas_call(
        _make_comb_body(True),
        grid=(N // _RB,),
        in_specs=_comb_specs() + [
            pl.BlockSpec((D, D), lambda i: (0, 0)),
            pl.BlockSpec((1, D), lambda i: (0, 0)),
            pl.BlockSpec((D, 2), lambda i: (0, 0)),
            pl.BlockSpec((1, 2), lambda i: (0, 0)),
        ],
        out_specs=pl.BlockSpec((_RB, 2), lambda i: (i, 0)),
        out_shape=jax.ShapeDtypeStruct((N, 2), jnp.float32),
    )(xs, sp, cp, W_root, W_rel, b, W_o1, b_o1, W_o2, b_o2)


def kernel(des, tweet, num_prop, cat_prop, edge_index, edge_type,
           W_cat, b_cat, W_in, b_in, W_rel, W_root, b_rgcn,
           W_o1, b_o1, W_o2, b_o2):
    del des, tweet, num_prop
    E = edge_index.shape[1]
    src = edge_index[0].astype(jnp.int32)
    dst = edge_index[1].astype(jnp.int32)
    et = edge_type.astype(jnp.int32)

    chunk = NSUB * SUP * BLK
    epad = (-E) % chunk
    if epad:
        src = jnp.concatenate([src, jnp.zeros((epad,), jnp.int32)])
        dst = jnp.concatenate([dst, jnp.full((epad,), DUMMY, jnp.int32)])
        et = jnp.concatenate([et, jnp.zeros((epad,), jnp.int32)])
    src2 = src.reshape(-1, BLK)
    dst2 = dst.reshape(-1, BLK)
    et2 = et.reshape(-1, BLK)
    n_sup = src2.shape[0] // (NSUB * SUP)

    b_cat2 = b_cat.reshape(1, D)
    b_in2 = b_in.reshape(1, D)
    b_rgcn2 = b_rgcn.reshape(1, D)
    b_o12 = b_o1.reshape(1, D)
    b_o22 = b_o2.reshape(1, 2)

    cnt = _make_cnt_kernel(n_sup)(dst2, et2)
    edge = _make_edge_kernel(n_sup)

    xs0 = _pre(cat_prop, W_cat, b_cat2, W_in, b_in2)
    sp1 = edge(xs0.reshape(2 * N, H), src2, dst2, et2)
    xs1 = _comb1(xs0, sp1, cnt, W_root, W_rel, b_rgcn2)
    sp2 = edge(xs1.reshape(2 * N, H), src2, dst2, et2)
    return _comb2(xs1, sp2, cnt, W_root, W_rel, b_rgcn2,
                  W_o1, b_o12, W_o2, b_o22)


# speedup vs baseline: 9.1814x; 1.0315x over previous
import jax
import jax.numpy as jnp
from jax import lax
from jax.experimental import pallas as pl
from jax.experimental.pallas import tpu as pltpu
from jax.experimental.pallas import tpu_sc as plsc

N = 10000
D = 128
H = 64
NREL = 2
NROW = 10112
NCORE = 2
NSUB = 16
BLK = 128
SUP = 16
DUMMY = N
NDEEP = 4


def _leaky(x):
    return jnp.where(x >= 0, x, 0.01 * x)


def _zero_acc(zbuf, acc, sid, rpt):
    zr = zbuf.shape[0]
    base = sid * rpt
    for k in range(rpt // zr):
        pltpu.sync_copy(zbuf, acc.at[pl.ds(base + k * zr, zr)])
    rem = rpt % zr
    if rem:
        pltpu.sync_copy(zbuf.at[pl.ds(0, rem)],
                        acc.at[pl.ds(base + (rpt // zr) * zr, rem)])


_SC_PARAMS = pltpu.CompilerParams(use_tc_tiling_on_sc=False)


def _make_edge_kernel(n_sup):
    rows_per_tile = n_sup * SUP
    sup2 = 2 * SUP
    rpt_acc = NREL * NROW // NSUB
    rpt_out = NROW // NSUB
    mesh = plsc.VectorSubcoreMesh(core_axis_name="c", subcore_axis_name="s")

    def body(xh, srch, dsth, eth, sp, acc, srcb, dstb, etb,
             rows_a, rows_b, rows_c, rows_d, zbuf, sem_a, sem_b, sem_c, sem_d):
        cid = lax.axis_index("c")
        sid = lax.axis_index("s")

        def _fill(i, carry):
            for g in range(H // 16):
                zbuf[i, pl.ds(g * 16, 16)] = jnp.zeros((16,), jnp.float32)
            return carry
        lax.fori_loop(0, 32, _fill, 0)

        _zero_acc(zbuf, acc, sid, rpt_acc)
        plsc.subcore_barrier()

        base0 = sid * rows_per_tile
        bufs = (rows_a, rows_b, rows_c, rows_d)
        sems = (sem_a, sem_b, sem_c, sem_d)
        src_off = cid * N

        def _sup(t, carry):
            base = base0 + t * sup2
            pltpu.sync_copy(srch.at[pl.ds(base, sup2)], srcb)
            pltpu.sync_copy(dsth.at[pl.ds(base, sup2)], dstb)
            pltpu.sync_copy(eth.at[pl.ds(base, sup2)], etb)

            nfly = NDEEP - 1
            def _gidx(j, c2):
                for g in range(BLK // 16):
                    sl = pl.ds(g * 16, 16)
                    srcb[j, sl] = srcb[j, sl] + src_off
                    dstb[j, sl] = etb[j, sl] * NROW + dstb[j, sl]
                return c2
            lax.fori_loop(0, nfly, _gidx, 0)

            descs = [pltpu.async_copy(xh.at[srcb.at[k]], bufs[k], sems[k])
                     for k in range(nfly)]

            def _gidx2(j, c2):
                return _gidx(j, c2)
            lax.fori_loop(nfly, sup2, _gidx2, 0)

            for j in range(sup2):
                descs[j].wait()
                if j + nfly < sup2:
                    descs.append(pltpu.async_copy(
                        xh.at[srcb.at[j + nfly]], bufs[(j + nfly) % NDEEP],
                        sems[(j + nfly) % NDEEP]))
                pltpu.sync_copy(bufs[j % NDEEP], acc.at[dstb.at[j]],
                                add=True)
            return carry
        lax.fori_loop(0, n_sup // 2, _sup, 0)
        plsc.subcore_barrier()

        for r in range(NREL):
            pltpu.sync_copy(
                acc.at[pl.ds(r * NROW + sid * rpt_out, rpt_out)],
                sp.at[cid, r, pl.ds(sid * rpt_out, rpt_out)])

    return pl.kernel(
        body,
        out_type=jax.ShapeDtypeStruct((NCORE, NREL, NROW, H), jnp.float32),
        mesh=mesh,
        compiler_params=_SC_PARAMS,
        scratch_types=[
            pltpu.VMEM_SHARED((NREL * NROW, H), jnp.float32),
            pltpu.VMEM((2 * SUP, BLK), jnp.int32),
            pltpu.VMEM((2 * SUP, BLK), jnp.int32),
            pltpu.VMEM((2 * SUP, BLK), jnp.int32),
            pltpu.VMEM((BLK, H), jnp.float32),
            pltpu.VMEM((BLK, H), jnp.float32),
            pltpu.VMEM((BLK, H), jnp.float32),
            pltpu.VMEM((BLK, H), jnp.float32),
            pltpu.VMEM((32, H), jnp.float32),
            pltpu.SemaphoreType.DMA,
            pltpu.SemaphoreType.DMA,
            pltpu.SemaphoreType.DMA,
            pltpu.SemaphoreType.DMA,
        ],
    )


def _make_cnt_kernel(n_sup):
    rows_per_tile = n_sup * SUP // NCORE
    rpt_acc = NREL * NROW // NSUB
    rpt_out = NROW // NSUB
    mesh = plsc.VectorSubcoreMesh(core_axis_name="c", subcore_axis_name="s")

    def body(dsth, eth, cnto, acc, dstb, etb, sidxb, ones, zbuf):
        cid = lax.axis_index("c")
        sid = lax.axis_index("s")

        def _fill(i, carry):
            ones[i, pl.ds(0, 16)] = jnp.ones((16,), jnp.float32)
            return carry
        lax.fori_loop(0, BLK, _fill, 0)

        def _fillz(i, carry):
            zbuf[i, pl.ds(0, 16)] = jnp.zeros((16,), jnp.float32)
            return carry
        lax.fori_loop(0, 64, _fillz, 0)

        _zero_acc(zbuf, acc, sid, rpt_acc)
        plsc.subcore_barrier()

        base0 = (cid * NSUB + sid) * rows_per_tile

        def _sup(t, carry):
            base = base0 + t * SUP
            pltpu.sync_copy(dsth.at[pl.ds(base, SUP)], dstb)
            pltpu.sync_copy(eth.at[pl.ds(base, SUP)], etb)

            def _sidx(j, c2):
                for g in range(BLK // 16):
                    sl = pl.ds(g * 16, 16)
                    sidxb[j, sl] = etb[j, sl] * NROW + dstb[j, sl]
                return c2
            lax.fori_loop(0, SUP, _sidx, 0)

            def _blk(j, c2):
                pltpu.sync_copy(ones, acc.at[sidxb.at[j]], add=True)
                return c2
            lax.fori_loop(0, SUP, _blk, 0)
            return carry
        lax.fori_loop(0, n_sup // NCORE, _sup, 0)
        plsc.subcore_barrier()

        for r in range(NREL):
            pltpu.sync_copy(
                acc.at[pl.ds(r * NROW + sid * rpt_out, rpt_out)],
                cnto.at[cid, r, pl.ds(sid * rpt_out, rpt_out)])

    return pl.kernel(
        body,
        out_type=jax.ShapeDtypeStruct((NCORE, NREL, NROW, 16), jnp.float32),
        mesh=mesh,
        compiler_params=_SC_PARAMS,
        scratch_types=[
            pltpu.VMEM_SHARED((NREL * NROW, 16), jnp.float32),
            pltpu.VMEM((SUP, BLK), jnp.int32),
            pltpu.VMEM((SUP, BLK), jnp.int32),
            pltpu.VMEM((SUP, BLK), jnp.int32),
            pltpu.VMEM((BLK, 16), jnp.float32),
            pltpu.VMEM((64, 16), jnp.float32),
        ],
    )


_RB = 2000


def _pre_body(cp, wcat, bcat, win, binp, out):
    c = _leaky(jnp.dot(cp[...], wcat[...],
                       preferred_element_type=jnp.float32) + bcat[...])
    x = _leaky(jnp.dot(c, win[...],
                       preferred_element_type=jnp.float32) + binp[...])
    out[0] = x[:, :H]
    out[1] = x[:, H:]


def _pre(cat_prop, W_cat, b_cat, W_in, b_in):
    return pl.pallas_call(
        _pre_body,
        grid=(N // _RB,),
        in_specs=[
            pl.BlockSpec((_RB, 11), lambda i: (i, 0)),
            pl.BlockSpec((11, D), lambda i: (0, 0)),
            pl.BlockSpec((1, D), lambda i: (0, 0)),
            pl.BlockSpec((D, D), lambda i: (0, 0)),
            pl.BlockSpec((1, D), lambda i: (0, 0)),
        ],
        out_specs=pl.BlockSpec((2, _RB, H), lambda i: (0, i, 0)),
        out_shape=jax.ShapeDtypeStruct((2, N, H), jnp.float32),
    )(cat_prop, W_cat, b_cat, W_in, b_in)


def _make_comb_body(head):
    def body(xs, sp, cp, wroot, wrel, b, *rest):
        if head:
            wo1, bo1, wo2, bo2, out = rest
        else:
            (out,) = rest
        x = jnp.concatenate([xs[0], xs[1]], axis=1)
        o = jnp.dot(x, wroot[...], preferred_element_type=jnp.float32) + b[...]
        for r in range(NREL):
            s = jnp.concatenate([sp[0, r], sp[1, r]], axis=1)
            cnt = cp[0, r, :, 0] + cp[1, r, :, 0]
            inv = 1.0 / jnp.maximum(cnt, 1.0)
            o = o + jnp.dot(s * inv[:, None], wrel[r],
                            preferred_element_type=jnp.float32)
        if head:
            y = _leaky(jnp.dot(o, wo1[...],
                               preferred_element_type=jnp.float32) + bo1[...])
            out[...] = jnp.dot(y, wo2[...],
                               preferred_element_type=jnp.float32) + bo2[...]
        else:
            out[0] = o[:, :H]
            out[1] = o[:, H:]
    return body


def _comb_specs():
    return [
        pl.BlockSpec((2, _RB, H), lambda i: (0, i, 0)),
        pl.BlockSpec((NCORE, NREL, _RB, H), lambda i: (0, 0, i, 0)),
        pl.BlockSpec((NCORE, NREL, _RB, 16), lambda i: (0, 0, i, 0)),
        pl.BlockSpec((D, D), lambda i: (0, 0)),
        pl.BlockSpec((NREL, D, D), lambda i: (0, 0, 0)),
        pl.BlockSpec((1, D), lambda i: (0, 0)),
    ]


def _comb1(xs, sp, cp, W_root, W_rel, b):
    return pl.pallas_call(
        _make_comb_body(False),
        grid=(N // _RB,),
        in_specs=_comb_specs(),
        out_specs=pl.BlockSpec((2, _RB, H), lambda i: (0, i, 0)),
        out_shape=jax.ShapeDtypeStruct((2, N, H), jnp.float32),
    )(xs, sp, cp, W_root, W_rel, b)


def _comb2(xs, sp, cp, W_root, W_rel, b, W_o1, b_o1, W_o2, b_o2):
    return pl.pallas_call(
        _make_comb_body(True),
        grid=(N // _RB,),
        in_specs=_comb_specs() + [
            pl.BlockSpec((D, D), lambda i: (0, 0)),
            pl.BlockSpec((1, D), lambda i: (0, 0)),
            pl.BlockSpec((D, 2), lambda i: (0, 0)),
            pl.BlockSpec((1, 2), lambda i: (0, 0)),
        ],
        out_specs=pl.BlockSpec((_RB, 2), lambda i: (i, 0)),
        out_shape=jax.ShapeDtypeStruct((N, 2), jnp.float32),
    )(xs, sp, cp, W_root, W_rel, b, W_o1, b_o1, W_o2, b_o2)


def kernel(des, tweet, num_prop, cat_prop, edge_index, edge_type,
           W_cat, b_cat, W_in, b_in, W_rel, W_root, b_rgcn,
           W_o1, b_o1, W_o2, b_o2):
    del des, tweet, num_prop
    E = edge_index.shape[1]
    src = edge_index[0].astype(jnp.int32)
    dst = edge_index[1].astype(jnp.int32)
    et = edge_type.astype(jnp.int32)

    chunk = NSUB * SUP * BLK
    epad = (-E) % chunk
    if epad:
        src = jnp.concatenate([src, jnp.zeros((epad,), jnp.int32)])
        dst = jnp.concatenate([dst, jnp.full((epad,), DUMMY, jnp.int32)])
        et = jnp.concatenate([et, jnp.zeros((epad,), jnp.int32)])
    src2 = src.reshape(-1, BLK)
    dst2 = dst.reshape(-1, BLK)
    et2 = et.reshape(-1, BLK)
    n_sup = src2.shape[0] // (NSUB * SUP)

    b_cat2 = b_cat.reshape(1, D)
    b_in2 = b_in.reshape(1, D)
    b_rgcn2 = b_rgcn.reshape(1, D)
    b_o12 = b_o1.reshape(1, D)
    b_o22 = b_o2.reshape(1, 2)

    cnt = _make_cnt_kernel(n_sup)(dst2, et2)
    edge = _make_edge_kernel(n_sup)

    xs0 = _pre(cat_prop, W_cat, b_cat2, W_in, b_in2)
    sp1 = edge(xs0.reshape(2 * N, H), src2, dst2, et2)
    xs1 = _comb1(xs0, sp1, cnt, W_root, W_rel, b_rgcn2)
    sp2 = edge(xs1.reshape(2 * N, H), src2, dst2, et2)
    return _comb2(xs1, sp2, cnt, W_root, W_rel, b_rgcn2,
                  W_o1, b_o12, W_o2, b_o22)


# speedup vs baseline: 9.3508x; 1.0184x over previous
import jax
import jax.numpy as jnp
from jax import lax
from jax.experimental import pallas as pl
from jax.experimental.pallas import tpu as pltpu
from jax.experimental.pallas import tpu_sc as plsc

N = 10000
D = 128
H = 64
NREL = 2
NROW = 10112
NCORE = 2
NSUB = 16
BLK = 128
SUP = 16
DUMMY = N
NDEEP = 4


def _leaky(x):
    return jnp.where(x >= 0, x, 0.01 * x)


def _zero_acc(zbuf, acc, sid, rpt):
    zr = zbuf.shape[0]
    base = sid * rpt
    for k in range(rpt // zr):
        pltpu.sync_copy(zbuf, acc.at[pl.ds(base + k * zr, zr)])
    rem = rpt % zr
    if rem:
        pltpu.sync_copy(zbuf.at[pl.ds(0, rem)],
                        acc.at[pl.ds(base + (rpt // zr) * zr, rem)])


_SC_PARAMS = pltpu.CompilerParams(use_tc_tiling_on_sc=False)


def _make_edge_kernel(n_sup):
    rows_per_tile = n_sup * SUP
    sup2 = 2 * SUP
    rpt_acc = NREL * NROW // NSUB
    rpt_out = NROW // NSUB
    mesh = plsc.VectorSubcoreMesh(core_axis_name="c", subcore_axis_name="s")

    def body(xh, srch, dsth, eth, sp, acc, srcb, dstb, etb,
             rows_a, rows_b, rows_c, rows_d, zbuf, sem_a, sem_b, sem_c, sem_d):
        cid = lax.axis_index("c")
        sid = lax.axis_index("s")

        def _fill(i, carry):
            for g in range(H // 16):
                zbuf[i, pl.ds(g * 16, 16)] = jnp.zeros((16,), jnp.float32)
            return carry
        lax.fori_loop(0, 32, _fill, 0)

        _zero_acc(zbuf, acc, sid, rpt_acc)
        plsc.subcore_barrier()

        base0 = sid * rows_per_tile
        bufs = (rows_a, rows_b, rows_c, rows_d)
        sems = (sem_a, sem_b, sem_c, sem_d)
        src_off = cid * N

        def _sup(t, carry):
            base = base0 + t * sup2
            pltpu.sync_copy(srch.at[pl.ds(base, sup2)], srcb)
            pltpu.sync_copy(dsth.at[pl.ds(base, sup2)], dstb)
            pltpu.sync_copy(eth.at[pl.ds(base, sup2)], etb)

            nfly = NDEEP - 1
            def _gidx(j, c2):
                for g in range(BLK // 16):
                    sl = pl.ds(g * 16, 16)
                    srcb[j, sl] = srcb[j, sl] + src_off
                    dstb[j, sl] = etb[j, sl] * NROW + dstb[j, sl]
                return c2
            lax.fori_loop(0, nfly, _gidx, 0)

            descs = [pltpu.async_copy(xh.at[srcb.at[k]], bufs[k], sems[k])
                     for k in range(nfly)]

            def _gidx2(j, c2):
                return _gidx(j, c2)
            lax.fori_loop(nfly, sup2, _gidx2, 0)

            for j in range(sup2):
                descs[j].wait()
                if j + nfly < sup2:
                    descs.append(pltpu.async_copy(
                        xh.at[srcb.at[j + nfly]], bufs[(j + nfly) % NDEEP],
                        sems[(j + nfly) % NDEEP]))
                pltpu.sync_copy(bufs[j % NDEEP], acc.at[dstb.at[j]],
                                add=True)
            return carry
        lax.fori_loop(0, n_sup // 2, _sup, 0)
        plsc.subcore_barrier()

        for r in range(NREL):
            pltpu.sync_copy(
                acc.at[pl.ds(r * NROW + sid * rpt_out, rpt_out)],
                sp.at[r, pl.ds(sid * rpt_out, rpt_out), pl.ds(cid * H, H)])

    return pl.kernel(
        body,
        out_type=jax.ShapeDtypeStruct((NREL, NROW, D), jnp.float32),
        mesh=mesh,
        compiler_params=_SC_PARAMS,
        scratch_types=[
            pltpu.VMEM_SHARED((NREL * NROW, H), jnp.float32),
            pltpu.VMEM((2 * SUP, BLK), jnp.int32),
            pltpu.VMEM((2 * SUP, BLK), jnp.int32),
            pltpu.VMEM((2 * SUP, BLK), jnp.int32),
            pltpu.VMEM((BLK, H), jnp.float32),
            pltpu.VMEM((BLK, H), jnp.float32),
            pltpu.VMEM((BLK, H), jnp.float32),
            pltpu.VMEM((BLK, H), jnp.float32),
            pltpu.VMEM((32, H), jnp.float32),
            pltpu.SemaphoreType.DMA,
            pltpu.SemaphoreType.DMA,
            pltpu.SemaphoreType.DMA,
            pltpu.SemaphoreType.DMA,
        ],
    )


def _make_cnt_kernel(n_sup):
    rows_per_tile = n_sup * SUP // NCORE
    rpt_acc = NREL * NROW // NSUB
    rpt_out = NROW // NSUB
    mesh = plsc.VectorSubcoreMesh(core_axis_name="c", subcore_axis_name="s")

    def body(dsth, eth, cnto, acc, dstb, etb, sidxb, ones, zbuf):
        cid = lax.axis_index("c")
        sid = lax.axis_index("s")

        def _fill(i, carry):
            ones[i, pl.ds(0, 16)] = jnp.ones((16,), jnp.float32)
            return carry
        lax.fori_loop(0, BLK, _fill, 0)

        def _fillz(i, carry):
            zbuf[i, pl.ds(0, 16)] = jnp.zeros((16,), jnp.float32)
            return carry
        lax.fori_loop(0, 64, _fillz, 0)

        _zero_acc(zbuf, acc, sid, rpt_acc)
        plsc.subcore_barrier()

        base0 = (cid * NSUB + sid) * rows_per_tile

        def _sup(t, carry):
            base = base0 + t * SUP
            pltpu.sync_copy(dsth.at[pl.ds(base, SUP)], dstb)
            pltpu.sync_copy(eth.at[pl.ds(base, SUP)], etb)

            def _sidx(j, c2):
                for g in range(BLK // 16):
                    sl = pl.ds(g * 16, 16)
                    sidxb[j, sl] = etb[j, sl] * NROW + dstb[j, sl]
                return c2
            lax.fori_loop(0, SUP, _sidx, 0)

            def _blk(j, c2):
                pltpu.sync_copy(ones, acc.at[sidxb.at[j]], add=True)
                return c2
            lax.fori_loop(0, SUP, _blk, 0)
            return carry
        lax.fori_loop(0, n_sup // NCORE, _sup, 0)
        plsc.subcore_barrier()

        for r in range(NREL):
            pltpu.sync_copy(
                acc.at[pl.ds(r * NROW + sid * rpt_out, rpt_out)],
                cnto.at[r, pl.ds(sid * rpt_out, rpt_out), pl.ds(cid * H, 16)])

    return pl.kernel(
        body,
        out_type=jax.ShapeDtypeStruct((NREL, NROW, D), jnp.float32),
        mesh=mesh,
        compiler_params=_SC_PARAMS,
        scratch_types=[
            pltpu.VMEM_SHARED((NREL * NROW, 16), jnp.float32),
            pltpu.VMEM((SUP, BLK), jnp.int32),
            pltpu.VMEM((SUP, BLK), jnp.int32),
            pltpu.VMEM((SUP, BLK), jnp.int32),
            pltpu.VMEM((BLK, 16), jnp.float32),
            pltpu.VMEM((64, 16), jnp.float32),
        ],
    )


_RB = 2000


def _pre_body(cp, wcat, bcat, win, binp, out):
    c = _leaky(jnp.dot(cp[...], wcat[...],
                       preferred_element_type=jnp.float32) + bcat[...])
    x = _leaky(jnp.dot(c, win[...],
                       preferred_element_type=jnp.float32) + binp[...])
    out[0] = x[:, :H]
    out[1] = x[:, H:]


def _pre(cat_prop, W_cat, b_cat, W_in, b_in):
    return pl.pallas_call(
        _pre_body,
        grid=(N // _RB,),
        in_specs=[
            pl.BlockSpec((_RB, 11), lambda i: (i, 0)),
            pl.BlockSpec((11, D), lambda i: (0, 0)),
            pl.BlockSpec((1, D), lambda i: (0, 0)),
            pl.BlockSpec((D, D), lambda i: (0, 0)),
            pl.BlockSpec((1, D), lambda i: (0, 0)),
        ],
        out_specs=pl.BlockSpec((2, _RB, H), lambda i: (0, i, 0)),
        out_shape=jax.ShapeDtypeStruct((2, N, H), jnp.float32),
    )(cat_prop, W_cat, b_cat, W_in, b_in)


def _make_comb_body(head):
    def body(xs, sp, cp, wroot, wrel, b, *rest):
        if head:
            wo1, bo1, wo2, bo2, out = rest
        else:
            (out,) = rest
        x = jnp.concatenate([xs[0], xs[1]], axis=1)
        o = jnp.dot(x, wroot[...], preferred_element_type=jnp.float32) + b[...]
        for r in range(NREL):
            s = sp[r]
            cnt = cp[r, :, 0] + cp[r, :, H]
            inv = 1.0 / jnp.maximum(cnt, 1.0)
            o = o + jnp.dot(s * inv[:, None], wrel[r],
                            preferred_element_type=jnp.float32)
        if head:
            y = _leaky(jnp.dot(o, wo1[...],
                               preferred_element_type=jnp.float32) + bo1[...])
            out[...] = jnp.dot(y, wo2[...],
                               preferred_element_type=jnp.float32) + bo2[...]
        else:
            out[0] = o[:, :H]
            out[1] = o[:, H:]
    return body


def _comb_specs():
    return [
        pl.BlockSpec((2, _RB, H), lambda i: (0, i, 0)),
        pl.BlockSpec((NREL, _RB, D), lambda i: (0, i, 0)),
        pl.BlockSpec((NREL, _RB, D), lambda i: (0, i, 0)),
        pl.BlockSpec((D, D), lambda i: (0, 0)),
        pl.BlockSpec((NREL, D, D), lambda i: (0, 0, 0)),
        pl.BlockSpec((1, D), lambda i: (0, 0)),
    ]


def _comb1(xs, sp, cp, W_root, W_rel, b):
    return pl.pallas_call(
        _make_comb_body(False),
        grid=(N // _RB,),
        in_specs=_comb_specs(),
        out_specs=pl.BlockSpec((2, _RB, H), lambda i: (0, i, 0)),
        out_shape=jax.ShapeDtypeStruct((2, N, H), jnp.float32),
    )(xs, sp, cp, W_root, W_rel, b)


def _comb2(xs, sp, cp, W_root, W_rel, b, W_o1, b_o1, W_o2, b_o2):
    return pl.pallas_call(
        _make_comb_body(True),
        grid=(N // _RB,),
        in_specs=_comb_specs() + [
            pl.BlockSpec((D, D), lambda i: (0, 0)),
            pl.BlockSpec((1, D), lambda i: (0, 0)),
            pl.BlockSpec((D, 2), lambda i: (0, 0)),
            pl.BlockSpec((1, 2), lambda i: (0, 0)),
        ],
        out_specs=pl.BlockSpec((_RB, 2), lambda i: (i, 0)),
        out_shape=jax.ShapeDtypeStruct((N, 2), jnp.float32),
    )(xs, sp, cp, W_root, W_rel, b, W_o1, b_o1, W_o2, b_o2)


def kernel(des, tweet, num_prop, cat_prop, edge_index, edge_type,
           W_cat, b_cat, W_in, b_in, W_rel, W_root, b_rgcn,
           W_o1, b_o1, W_o2, b_o2):
    del des, tweet, num_prop
    E = edge_index.shape[1]
    src = edge_index[0].astype(jnp.int32)
    dst = edge_index[1].astype(jnp.int32)
    et = edge_type.astype(jnp.int32)

    chunk = NSUB * SUP * BLK
    epad = (-E) % chunk
    if epad:
        src = jnp.concatenate([src, jnp.zeros((epad,), jnp.int32)])
        dst = jnp.concatenate([dst, jnp.full((epad,), DUMMY, jnp.int32)])
        et = jnp.concatenate([et, jnp.zeros((epad,), jnp.int32)])
    src2 = src.reshape(-1, BLK)
    dst2 = dst.reshape(-1, BLK)
    et2 = et.reshape(-1, BLK)
    n_sup = src2.shape[0] // (NSUB * SUP)

    b_cat2 = b_cat.reshape(1, D)
    b_in2 = b_in.reshape(1, D)
    b_rgcn2 = b_rgcn.reshape(1, D)
    b_o12 = b_o1.reshape(1, D)
    b_o22 = b_o2.reshape(1, 2)

    cnt = _make_cnt_kernel(n_sup)(dst2, et2)
    edge = _make_edge_kernel(n_sup)

    xs0 = _pre(cat_prop, W_cat, b_cat2, W_in, b_in2)
    sp1 = edge(xs0.reshape(2 * N, H), src2, dst2, et2)
    xs1 = _comb1(xs0, sp1, cnt, W_root, W_rel, b_rgcn2)
    sp2 = edge(xs1.reshape(2 * N, H), src2, dst2, et2)
    return _comb2(xs1, sp2, cnt, W_root, W_rel, b_rgcn2,
                  W_o1, b_o12, W_o2, b_o22)
